# baseline (speedup 1.0000x reference)
import functools
import math

import jax
import jax.numpy as jnp
from jax import lax
from jax.experimental import pallas as pl
from jax.experimental.pallas import tpu as pltpu
from jax.experimental.pallas import tpu_sc as plsc

_N = 4096
_V = 32000
_SMOOTHING = 0.1
_BR = 128

_S = _SMOOTHING / (_V - 3)
_CONF = 1.0 - _SMOOTHING
_C2 = (_V - 2) * _S * math.log(_S)
_C3 = (_V - 3) * _S * math.log(_S) + _CONF * math.log(_CONF)

_NC = 2
_NS = 16
_NW = _NC * _NS
_NSC = 1024
_RPS = _NSC // _NW
_NBUF = 3
_CHUNKS = _V // 16
_UNROLL = 5


@functools.partial(
    pl.kernel,
    mesh=plsc.VectorSubcoreMesh(core_axis_name="c", subcore_axis_name="s"),
    out_type=[
        jax.ShapeDtypeStruct((_NSC * 16,), jnp.float32),
        jax.ShapeDtypeStruct((_NSC * 16,), jnp.float32),
    ],
    scratch_types=(
        [pltpu.VMEM((_V,), jnp.float32) for _ in range(_NBUF)]
        + [pltpu.VMEM((_RPS,), jnp.int32),
           pltpu.VMEM((_RPS * 16,), jnp.float32),
           pltpu.VMEM((_RPS * 16,), jnp.float32)]
        + [pltpu.SemaphoreType.DMA for _ in range(_NBUF)]
    ),
)
def _sc_rows(x_hbm, tgt_hbm, sums_hbm, gv_hbm, *refs):
    bufs = refs[:_NBUF]
    t_v, sums_v, gv_v = refs[_NBUF:_NBUF + 3]
    sems = refs[_NBUF + 3:]

    wid = lax.axis_index("s") * _NC + lax.axis_index("c")
    row0 = wid * _RPS
    pltpu.sync_copy(tgt_hbm.at[pl.ds(row0, _RPS)], t_v)

    handles = [None] * _RPS
    for b in range(min(_NBUF, _RPS)):
        handles[b] = pltpu.async_copy(x_hbm.at[row0 + b], bufs[b], sems[b])

    lane = lax.iota(jnp.int32, 16)
    per_iter = 16 * _UNROLL

    for r in range(_RPS):
        b = r % _NBUF
        handles[r].wait()
        buf = bufs[b]

        def body(i, accs):
            base = i * per_iter
            return tuple(a + buf[pl.ds(base + k * 16, 16)]
                         for k, a in enumerate(accs))

        zeros = jnp.zeros((16,), jnp.float32)
        accs = lax.fori_loop(0, _CHUNKS // _UNROLL, body, (zeros,) * _UNROLL)
        acc = accs[0]
        for a in accs[1:]:
            acc = acc + a
        chunk0 = buf[pl.ds(0, 16)]
        acc = acc - jnp.where(lane < 2, chunk0, 0.0)
        sums_v[pl.ds(r * 16, 16)] = acc

        tr = t_v[pl.ds((r // 16) * 16, 16)][r % 16]
        win16 = buf[pl.ds((tr // 16) * 16, 16)]
        gv_v[pl.ds(r * 16, 16)] = jnp.where(lane == tr % 16, win16, 0.0)

        nxt = r + _NBUF
        if nxt < _RPS:
            handles[nxt] = pltpu.async_copy(x_hbm.at[row0 + nxt], buf, sems[b])

    pltpu.sync_copy(sums_v, sums_hbm.at[pl.ds(row0 * 16, _RPS * 16)])
    pltpu.sync_copy(gv_v, gv_hbm.at[pl.ds(row0 * 16, _RPS * 16)])


def _loss_block(ts_ref, x_ref, t_ref, out_ref, win_ref):
    i = pl.program_id(0)
    x = x_ref[...]
    t = t_ref[0]

    rs = jnp.sum(x, axis=1, keepdims=True)
    s_i = rs - x[:, 0:1] - x[:, 1:2]
    reg = t >= 2
    dense = jnp.sum(jnp.where(t != 0, s_i, 0.0))
    n_reg = jnp.sum(reg.astype(jnp.float32))
    n_one = jnp.sum((t == 1).astype(jnp.float32))

    for r in range(_BR):
        c0 = pl.multiple_of((ts_ref[0, 0, r] // 128) * 128, 128)
        win_ref[pl.ds(r, 1), :] = x_ref[pl.ds(r, 1), pl.ds(c0, 128)]
    lane = t % 128
    col = lax.broadcasted_iota(jnp.int32, (_BR, 128), 1)
    gath = jnp.sum(jnp.where((col == lane) & reg, win_ref[...], 0.0))

    partial = jnp.reshape(_C3 * n_reg + _C2 * n_one
                          - _S * dense - (_CONF - _S) * gath, (1, 1))

    @pl.when(i == 0)
    def _init():
        out_ref[...] = partial

    @pl.when(i != 0)
    def _acc():
        out_ref[...] += partial


def _combine(part_ref, sums_ref, gv_ref, t_ref, out_ref):
    t = t_ref[...]
    s_i = jnp.sum(sums_ref[...], axis=1, keepdims=True)
    g = jnp.sum(gv_ref[...], axis=1, keepdims=True)
    reg = t >= 2
    dense = jnp.sum(jnp.where(t != 0, s_i, 0.0))
    gath = jnp.sum(jnp.where(reg, g, 0.0))
    n_reg = jnp.sum(reg.astype(jnp.float32))
    n_one = jnp.sum((t == 1).astype(jnp.float32))
    out_ref[...] = part_ref[...] + jnp.reshape(
        _C3 * n_reg + _C2 * n_one - _S * dense - (_CONF - _S) * gath, (1, 1))


def kernel(model_output_dist, target_sequence):
    n, v = model_output_dist.shape
    t = target_sequence.astype(jnp.int32)

    sums, gv = _sc_rows(model_output_dist, t)

    nb = (n - _NSC) // _BR
    t_tc = t[_NSC:]
    part = pl.pallas_call(
        _loss_block,
        grid=(nb,),
        in_specs=[
            pl.BlockSpec((1, 1, _BR), lambda i: (i, 0, 0),
                         memory_space=pltpu.SMEM),
            pl.BlockSpec((_BR, v), lambda i: (i + _NSC // _BR, 0)),
            pl.BlockSpec((1, _BR, 1), lambda i: (i, 0, 0)),
        ],
        out_specs=pl.BlockSpec((1, 1), lambda i: (0, 0)),
        out_shape=jax.ShapeDtypeStruct((1, 1), jnp.float32),
        scratch_shapes=[pltpu.VMEM((_BR, 128), jnp.float32)],
    )(t_tc.reshape(nb, 1, _BR), model_output_dist,
      t_tc.reshape(nb, _BR, 1))

    out = pl.pallas_call(
        _combine,
        out_shape=jax.ShapeDtypeStruct((1, 1), jnp.float32),
    )(part, sums.reshape(_NSC, 16), gv.reshape(_NSC, 16),
      t[:_NSC].reshape(_NSC, 1))
    return out[0, 0]


# speedup vs baseline: 1.0222x; 1.0222x over previous
import functools
import math

import jax
import jax.numpy as jnp
from jax import lax
from jax.experimental import pallas as pl
from jax.experimental.pallas import tpu as pltpu
from jax.experimental.pallas import tpu_sc as plsc

_N = 4096
_V = 32000
_SMOOTHING = 0.1
_BR = 128

_S = _SMOOTHING / (_V - 3)
_CONF = 1.0 - _SMOOTHING
_C2 = (_V - 2) * _S * math.log(_S)
_C3 = (_V - 3) * _S * math.log(_S) + _CONF * math.log(_CONF)

_NC = 2
_NS = 16
_NW = _NC * _NS
_NSC = 512
_RPS = _NSC // _NW
_NBUF = 3
_CHUNKS = _V // 16
_UNROLL = 5


@functools.partial(
    pl.kernel,
    mesh=plsc.VectorSubcoreMesh(core_axis_name="c", subcore_axis_name="s"),
    out_type=[
        jax.ShapeDtypeStruct((_NSC * 16,), jnp.float32),
        jax.ShapeDtypeStruct((_NSC * 16,), jnp.float32),
    ],
    scratch_types=(
        [pltpu.VMEM((_V,), jnp.float32) for _ in range(_NBUF)]
        + [pltpu.VMEM((_RPS,), jnp.int32),
           pltpu.VMEM((_RPS * 16,), jnp.float32),
           pltpu.VMEM((_RPS * 16,), jnp.float32)]
        + [pltpu.SemaphoreType.DMA for _ in range(_NBUF)]
    ),
)
def _sc_rows(x_hbm, tgt_hbm, sums_hbm, gv_hbm, *refs):
    bufs = refs[:_NBUF]
    t_v, sums_v, gv_v = refs[_NBUF:_NBUF + 3]
    sems = refs[_NBUF + 3:]

    wid = lax.axis_index("s") * _NC + lax.axis_index("c")
    row0 = wid * _RPS
    pltpu.sync_copy(tgt_hbm.at[pl.ds(row0, _RPS)], t_v)

    handles = [None] * _RPS
    for b in range(min(_NBUF, _RPS)):
        handles[b] = pltpu.async_copy(x_hbm.at[row0 + b], bufs[b], sems[b])

    lane = lax.iota(jnp.int32, 16)
    per_iter = 16 * _UNROLL

    for r in range(_RPS):
        b = r % _NBUF
        handles[r].wait()
        buf = bufs[b]

        def body(i, accs):
            base = i * per_iter
            return tuple(a + buf[pl.ds(base + k * 16, 16)]
                         for k, a in enumerate(accs))

        zeros = jnp.zeros((16,), jnp.float32)
        accs = lax.fori_loop(0, _CHUNKS // _UNROLL, body, (zeros,) * _UNROLL)
        acc = accs[0]
        for a in accs[1:]:
            acc = acc + a
        chunk0 = buf[pl.ds(0, 16)]
        acc = acc - jnp.where(lane < 2, chunk0, 0.0)
        sums_v[pl.ds(r * 16, 16)] = acc

        tr = t_v[pl.ds((r // 16) * 16, 16)][r % 16]
        win16 = buf[pl.ds((tr // 16) * 16, 16)]
        gv_v[pl.ds(r * 16, 16)] = jnp.where(lane == tr % 16, win16, 0.0)

        nxt = r + _NBUF
        if nxt < _RPS:
            handles[nxt] = pltpu.async_copy(x_hbm.at[row0 + nxt], buf, sems[b])

    pltpu.sync_copy(sums_v, sums_hbm.at[pl.ds(row0 * 16, _RPS * 16)])
    pltpu.sync_copy(gv_v, gv_hbm.at[pl.ds(row0 * 16, _RPS * 16)])


def _loss_block(ts_ref, x_ref, t_ref, out_ref, win_ref):
    i = pl.program_id(0)
    x = x_ref[...]
    t = t_ref[0]

    rs = jnp.sum(x, axis=1, keepdims=True)
    s_i = rs - x[:, 0:1] - x[:, 1:2]
    reg = t >= 2
    dense = jnp.sum(jnp.where(t != 0, s_i, 0.0))
    n_reg = jnp.sum(reg.astype(jnp.float32))
    n_one = jnp.sum((t == 1).astype(jnp.float32))

    for r in range(_BR):
        c0 = pl.multiple_of((ts_ref[0, 0, r] // 128) * 128, 128)
        win_ref[pl.ds(r, 1), :] = x_ref[pl.ds(r, 1), pl.ds(c0, 128)]
    lane = t % 128
    col = lax.broadcasted_iota(jnp.int32, (_BR, 128), 1)
    gath = jnp.sum(jnp.where((col == lane) & reg, win_ref[...], 0.0))

    partial = jnp.reshape(_C3 * n_reg + _C2 * n_one
                          - _S * dense - (_CONF - _S) * gath, (1, 1))

    @pl.when(i == 0)
    def _init():
        out_ref[...] = partial

    @pl.when(i != 0)
    def _acc():
        out_ref[...] += partial


def _combine(part_ref, sums_ref, gv_ref, t_ref, out_ref):
    t = t_ref[...]
    s_i = jnp.sum(sums_ref[...], axis=1, keepdims=True)
    g = jnp.sum(gv_ref[...], axis=1, keepdims=True)
    reg = t >= 2
    dense = jnp.sum(jnp.where(t != 0, s_i, 0.0))
    gath = jnp.sum(jnp.where(reg, g, 0.0))
    n_reg = jnp.sum(reg.astype(jnp.float32))
    n_one = jnp.sum((t == 1).astype(jnp.float32))
    out_ref[...] = part_ref[...] + jnp.reshape(
        _C3 * n_reg + _C2 * n_one - _S * dense - (_CONF - _S) * gath, (1, 1))


def kernel(model_output_dist, target_sequence):
    n, v = model_output_dist.shape
    t = target_sequence.astype(jnp.int32)

    sums, gv = _sc_rows(model_output_dist, t)

    nb = (n - _NSC) // _BR
    t_tc = t[_NSC:]
    part = pl.pallas_call(
        _loss_block,
        grid=(nb,),
        in_specs=[
            pl.BlockSpec((1, 1, _BR), lambda i: (i, 0, 0),
                         memory_space=pltpu.SMEM),
            pl.BlockSpec((_BR, v), lambda i: (i + _NSC // _BR, 0)),
            pl.BlockSpec((1, _BR, 1), lambda i: (i, 0, 0)),
        ],
        out_specs=pl.BlockSpec((1, 1), lambda i: (0, 0)),
        out_shape=jax.ShapeDtypeStruct((1, 1), jnp.float32),
        scratch_shapes=[pltpu.VMEM((_BR, 128), jnp.float32)],
    )(t_tc.reshape(nb, 1, _BR), model_output_dist,
      t_tc.reshape(nb, _BR, 1))

    out = pl.pallas_call(
        _combine,
        out_shape=jax.ShapeDtypeStruct((1, 1), jnp.float32),
    )(part, sums.reshape(_NSC, 16), gv.reshape(_NSC, 16),
      t[:_NSC].reshape(_NSC, 1))
    return out[0, 0]


# speedup vs baseline: 1.1739x; 1.1485x over previous
import math

import jax
import jax.numpy as jnp
from jax import lax
from jax.experimental import pallas as pl
from jax.experimental.pallas import tpu as pltpu

_N = 4096
_V = 32000
_SMOOTHING = 0.1
_BR = 64

_S = _SMOOTHING / (_V - 3)
_CONF = 1.0 - _SMOOTHING
_C2 = (_V - 2) * _S * math.log(_S)
_C3 = (_V - 3) * _S * math.log(_S) + _CONF * math.log(_CONF)


def _loss_block(ts_ref, x_ref, t_ref, out_ref, win_ref):
    i = pl.program_id(0)
    x = x_ref[...]
    t = t_ref[0]

    rs = jnp.sum(x, axis=1, keepdims=True)
    s_i = rs - x[:, 0:1] - x[:, 1:2]
    reg = t >= 2
    dense = jnp.sum(jnp.where(t != 0, s_i, 0.0))
    n_reg = jnp.sum(reg.astype(jnp.float32))
    n_one = jnp.sum((t == 1).astype(jnp.float32))

    for r in range(_BR):
        c0 = pl.multiple_of((ts_ref[0, 0, r] // 128) * 128, 128)
        win_ref[pl.ds(r, 1), :] = x_ref[pl.ds(r, 1), pl.ds(c0, 128)]
    lane = t % 128
    col = lax.broadcasted_iota(jnp.int32, (_BR, 128), 1)
    gath = jnp.sum(jnp.where((col == lane) & reg, win_ref[...], 0.0))

    partial = jnp.reshape(_C3 * n_reg + _C2 * n_one
                          - _S * dense - (_CONF - _S) * gath, (1, 1))

    @pl.when(i == 0)
    def _init():
        out_ref[...] = partial

    @pl.when(i != 0)
    def _acc():
        out_ref[...] += partial


def kernel(model_output_dist, target_sequence):
    n, v = model_output_dist.shape
    nb = n // _BR
    t = target_sequence.astype(jnp.int32)
    out = pl.pallas_call(
        _loss_block,
        grid=(nb,),
        in_specs=[
            pl.BlockSpec((1, 1, _BR), lambda i: (i, 0, 0),
                         memory_space=pltpu.SMEM),
            pl.BlockSpec((_BR, v), lambda i: (i, 0)),
            pl.BlockSpec((1, _BR, 1), lambda i: (i, 0, 0)),
        ],
        out_specs=pl.BlockSpec((1, 1), lambda i: (0, 0)),
        out_shape=jax.ShapeDtypeStruct((1, 1), jnp.float32),
        scratch_shapes=[pltpu.VMEM((_BR, 128), jnp.float32)],
    )(t.reshape(nb, 1, _BR), model_output_dist, t.reshape(nb, _BR, 1))
    return out[0, 0]
